# Initial kernel scaffold; baseline (speedup 1.0000x reference)
import jax
import jax.numpy as jnp
from jax.experimental import pallas as pl


def kernel(points, t, hash_s, hash_d, sp0, tp0, sp1, tp1, W, b, gamma, beta):
    raise NotImplementedError("write your pallas kernel here")


# speedup vs baseline: 13.8211x; 13.8211x over previous
import functools
import math

import jax
import jax.numpy as jnp
import numpy as np
from jax import lax
from jax.experimental import pallas as pl
from jax.experimental.pallas import tpu as pltpu
from jax.experimental.pallas import tpu_sc as plsc

_HSZ = 524288
_MASK = _HSZ - 1
_LH = 8
_BASE, _MAXR = 16, 512
_GROWTH = math.exp(math.log(_MAXR / _BASE) / (_LH - 1))
_LEVELS = [int(math.floor(_BASE * _GROWTH ** l + 1e-6)) for l in range(_LH)]
_TIME_RES = 25
_FRAMES = 6
_P1 = np.int32(2654435761 - (1 << 32))
_P2 = np.int32(805459861)
_P3 = np.int32(3674653429 - (1 << 32))

_NW = 32
_CH = 64
_NG = _CH // 16


def _sc_features(xs, ys, zs, params, hsf, hdf, sp0f, sp1f, tp0f, tp1f, n,
                 interpret):
    ch = _CH
    pw = n // _NW
    nch = pw // ch
    f32 = jnp.float32
    i32 = jnp.int32
    mesh = plsc.VectorSubcoreMesh(core_axis_name="c", subcore_axis_name="s",
                                  num_cores=2, num_subcores=16)

    @functools.partial(
        pl.kernel,
        out_type=jax.ShapeDtypeStruct((n, 64), f32),
        mesh=mesh,
        interpret=interpret,
        compiler_params=pltpu.CompilerParams(
            needs_layout_passes=False, use_tc_tiling_on_sc=False),
        scratch_types=[
            pltpu.VMEM((ch,), f32),
            pltpu.VMEM((ch,), f32),
            pltpu.VMEM((ch,), f32),
            pltpu.VMEM((16,), f32),
            pltpu.VMEM((2048,), i32),
            pltpu.VMEM((2048,), i32),
            pltpu.VMEM((2048,), i32),
            pltpu.VMEM((2048,), i32),
            pltpu.VMEM((2048,), i32),
            pltpu.VMEM((2048,), i32),
            pltpu.VMEM((2048, 8), f32),
            pltpu.VMEM((2048, 8), f32),
            pltpu.VMEM((768,), i32),
            pltpu.VMEM((768,), i32),
            pltpu.VMEM((768,), i32),
            pltpu.VMEM((768,), i32),
            pltpu.VMEM((768, 8), f32),
            pltpu.VMEM((768, 8), f32),
            pltpu.VMEM((ch, 64), f32),
            pltpu.SemaphoreType.DMA,
            pltpu.SemaphoreType.DMA,
            pltpu.SemaphoreType.DMA,
            pltpu.SemaphoreType.DMA,
        ],
    )
    def feat_kernel(xs_h, ys_h, zs_h, par_h, hs_h, hd_h, s0_h, s1_h, t0_h,
                    t1_h, out_h,
                    xv, yv, zv, pv, sidx0, sidx1, didx0, didx1, didx2, didx3,
                    hrow0, hrow1, pidx0, pidx1, pidxt0, pidxt1, prowA, prowB,
                    featb, semA, semB, semC, semD):
        wid = lax.axis_index("s") * 2 + lax.axis_index("c")
        lanes = lax.iota(i32, 16)
        zeros_i = jnp.zeros((16,), i32)
        ones_i = jnp.full((16,), 1, i32)

        pltpu.sync_copy(par_h, pv)
        tv = pv[...]
        pt = tv * np.float32(_TIME_RES)
        ti = pt.astype(i32)
        ftv = pt - ti.astype(f32)
        wt0 = 1.0 - ftv
        wt1 = ftv
        ht0 = ti * _P3
        ht1 = ht0 + _P3
        bt = tv * np.float32(_TIME_RES - 1)
        bti = jnp.minimum(jnp.maximum(bt.astype(i32), 0),
                          np.int32(_TIME_RES - 2))
        fbt = bt - bti.astype(f32)
        wbt0 = 1.0 - fbt
        wbt1 = fbt

        sidxs = (sidx0, sidx1)
        didxs = (didx0, didx1, didx2, didx3)

        def cfrac(c, rm1f, rmax):
            p = c * rm1f
            ci = jnp.minimum(p.astype(i32), np.int32(rmax))
            ci = jnp.maximum(ci, np.int32(0))
            return ci, p - ci.astype(f32)

        def gen_spatial_planes(g, x, y, z, R, pref):
            rm1 = np.float32(R - 1)
            xi, _ = cfrac(x, rm1, R - 2)
            yi, _ = cfrac(y, rm1, R - 2)
            zi, _ = cfrac(z, rm1, R - 2)
            coords = ((xi, yi), (xi, zi), (yi, zi))
            for k in range(3):
                ai, bi = coords[k]
                off = np.int32(k * R * R)
                for da in (0, 1):
                    for db in (0, 1):
                        cor = da * 2 + db
                        row = (ai + da) * np.int32(R) + (bi + db) + off
                        pref[pl.ds(k * 256 + cor * 64 + g * 16, 16)] = row

        def gen_temporal_planes(g, x, y, z, R1, pref):
            rm1 = np.float32(R1 - 1)
            coords = (x, y, z)
            for k in range(3):
                ai, _ = cfrac(coords[k], rm1, R1 - 2)
                off = np.int32(k * R1 * _TIME_RES)
                for da in (0, 1):
                    for db in (0, 1):
                        cor = da * 2 + db
                        row = ((ai + da) * np.int32(_TIME_RES)
                               + (bti + db) + off)
                        pref[pl.ds(k * 256 + cor * 64 + g * 16, 16)] = row

        def interp_spatial_planes(g, x, y, z, R, prow, col0):
            rm1 = np.float32(R - 1)
            _, fx = cfrac(x, rm1, R - 2)
            _, fy = cfrac(y, rm1, R - 2)
            _, fz = cfrac(z, rm1, R - 2)
            fr = (fx, fy, fz)
            pairs = ((0, 1), (0, 2), (1, 2))
            ws = []
            for k in range(3):
                fa, fb = fr[pairs[k][0]], fr[pairs[k][1]]
                ws.append(((1.0 - fa) * (1.0 - fb), (1.0 - fa) * fb,
                           fa * (1.0 - fb), fa * fb))
            rowv = g * 16 + lanes
            for chn in range(8):
                cv = jnp.full((16,), chn, i32)
                prod = None
                for k in range(3):
                    acc = None
                    for cor in range(4):
                        ridx = k * 256 + cor * 64 + g * 16 + lanes
                        fv = plsc.load_gather(prow, [ridx, cv])
                        term = ws[k][cor] * fv
                        acc = term if acc is None else acc + term
                    prod = acc if prod is None else prod * acc
                plsc.store_scatter(
                    featb, [rowv, jnp.full((16,), col0 + chn, i32)], prod)

        def interp_temporal_planes(g, x, y, z, R1, prow, col0):
            rm1 = np.float32(R1 - 1)
            coords = (x, y, z)
            ws = []
            for k in range(3):
                _, fa = cfrac(coords[k], rm1, R1 - 2)
                ws.append(((1.0 - fa) * wbt0, (1.0 - fa) * wbt1,
                           fa * wbt0, fa * wbt1))
            rowv = g * 16 + lanes
            for chn in range(8):
                cv = jnp.full((16,), chn, i32)
                prod = None
                for k in range(3):
                    acc = None
                    for cor in range(4):
                        ridx = k * 256 + cor * 64 + g * 16 + lanes
                        fv = plsc.load_gather(prow, [ridx, cv])
                        term = ws[k][cor] * fv
                        acc = term if acc is None else acc + term
                    prod = acc if prod is None else prod * acc
                plsc.store_scatter(
                    featb, [rowv, jnp.full((16,), col0 + chn, i32)], prod)

        def corner_hashes(x, y, z, l):
            r = np.float32(_LEVELS[l])
            px = x * r
            py = y * r
            pz = z * r
            xi = px.astype(i32)
            yi = py.astype(i32)
            zi = pz.astype(i32)
            fx = px - xi.astype(f32)
            fy = py - yi.astype(f32)
            fz = pz - zi.astype(f32)
            hy0 = yi * _P1
            hz0 = zi * _P2
            hxs = (xi, xi + 1)
            hys = (hy0, hy0 + _P1)
            hzs = (hz0, hz0 + _P2)
            hsps = []
            for dx in (0, 1):
                for dy in (0, 1):
                    hxy = hxs[dx] ^ hys[dy]
                    for dz in (0, 1):
                        hsps.append(hxy ^ hzs[dz])
            return hsps, (1.0 - fx, fx), (1.0 - fy, fy), (1.0 - fz, fz)

        def interp_static_level(g, x, y, z, l, hrow):
            hsps, wxs, wys, wzs = corner_hashes(x, y, z, l)
            rowv = g * 16 + lanes
            acc0 = jnp.zeros((16,), f32)
            acc1 = jnp.zeros((16,), f32)
            for dx in (0, 1):
                for dy in (0, 1):
                    wxy = wxs[dx] * wys[dy]
                    for dz in (0, 1):
                        ci = dx * 4 + dy * 2 + dz
                        ridx = (l % 4) * 512 + g * 128 + ci * 16 + lanes
                        col0 = (hsps[ci] & 3) * 2
                        f0 = plsc.load_gather(hrow, [ridx, col0])
                        f1 = plsc.load_gather(hrow, [ridx, col0 + 1])
                        w = wxy * wzs[dz]
                        acc0 = acc0 + w * f0
                        acc1 = acc1 + w * f1
            plsc.store_scatter(featb, [rowv, jnp.full((16,), 32 + 2 * l, i32)],
                               acc0)
            plsc.store_scatter(featb, [rowv, jnp.full((16,), 33 + 2 * l, i32)],
                               acc1)

        def interp_dynamic_level(g, x, y, z, l, hrow):
            hsps, wxs, wys, wzs = corner_hashes(x, y, z, l)
            rowv = g * 16 + lanes
            acc0 = jnp.zeros((16,), f32)
            acc1 = jnp.zeros((16,), f32)
            for dx in (0, 1):
                for dy in (0, 1):
                    wxy = wxs[dx] * wys[dy]
                    for dz in (0, 1):
                        ci = dx * 4 + dy * 2 + dz
                        rbase = (l % 2) * 1024 + g * 256 + ci * 32
                        r0 = rbase + lanes
                        r1 = rbase + 16 + lanes
                        c0 = ((hsps[ci] ^ ht0) & 3) * 2
                        c1 = ((hsps[ci] ^ ht1) & 3) * 2
                        fa0 = plsc.load_gather(hrow, [r0, c0])
                        fa1 = plsc.load_gather(hrow, [r0, c0 + 1])
                        fb0 = plsc.load_gather(hrow, [r1, c1])
                        fb1 = plsc.load_gather(hrow, [r1, c1 + 1])
                        w = wxy * wzs[dz]
                        acc0 = acc0 + w * (wt0 * fa0 + wt1 * fb0)
                        acc1 = acc1 + w * (wt0 * fa1 + wt1 * fb1)
            plsc.store_scatter(featb, [rowv, jnp.full((16,), 48 + 2 * l, i32)],
                               acc0)
            plsc.store_scatter(featb, [rowv, jnp.full((16,), 49 + 2 * l, i32)],
                               acc1)

        def chunk_body(cidx, carry):
            base = wid * pw + cidx * ch
            pltpu.sync_copy(xs_h.at[pl.ds(base, ch)], xv)
            pltpu.sync_copy(ys_h.at[pl.ds(base, ch)], yv)
            pltpu.sync_copy(zs_h.at[pl.ds(base, ch)], zv)

            def gen(g, c2):
                x = xv[pl.ds(g * 16, 16)]
                y = yv[pl.ds(g * 16, 16)]
                z = zv[pl.ds(g * 16, 16)]
                for l in range(_LH):
                    r = np.float32(_LEVELS[l])
                    xi = (x * r).astype(i32)
                    yi = (y * r).astype(i32)
                    zi = (z * r).astype(i32)
                    hxs = (xi, xi + 1)
                    hy0 = yi * _P1
                    hys = (hy0, hy0 + _P1)
                    hz0 = zi * _P2
                    hzs = (hz0, hz0 + _P2)
                    loff8 = np.int32(l * (_HSZ // 4))
                    sref = sidxs[l // 4]
                    sbase = (l % 4) * 512 + g * 128
                    dref = didxs[l // 2]
                    dbase = (l % 2) * 1024 + g * 256
                    for dx in (0, 1):
                        for dy in (0, 1):
                            hxy = hxs[dx] ^ hys[dy]
                            for dz in (0, 1):
                                ci = dx * 4 + dy * 2 + dz
                                hsp = hxy ^ hzs[dz]
                                sref[pl.ds(sbase + ci * 16, 16)] = (
                                    ((hsp & _MASK) >> 2) + loff8)
                                dref[pl.ds(dbase + ci * 32, 16)] = (
                                    (((hsp ^ ht0) & _MASK) >> 2) + loff8)
                                dref[pl.ds(dbase + ci * 32 + 16, 16)] = (
                                    (((hsp ^ ht1) & _MASK) >> 2) + loff8)
                gen_spatial_planes(g, x, y, z, 64, pidx0)
                gen_spatial_planes(g, x, y, z, 128, pidx1)
                gen_temporal_planes(g, x, y, z, 64, pidxt0)
                gen_temporal_planes(g, x, y, z, 128, pidxt1)
                return c2

            lax.fori_loop(0, _NG, gen, 0)

            cPA = pltpu.async_copy(s0_h.at[pidx0], prowA, semC)
            cPB = pltpu.async_copy(s1_h.at[pidx1], prowB, semD)
            cH0 = pltpu.async_copy(hs_h.at[sidx0], hrow0, semA)
            cH1 = pltpu.async_copy(hs_h.at[sidx1], hrow1, semB)

            cPA.wait()

            def isp0(g, c2):
                x = xv[pl.ds(g * 16, 16)]
                y = yv[pl.ds(g * 16, 16)]
                z = zv[pl.ds(g * 16, 16)]
                interp_spatial_planes(g, x, y, z, 64, prowA, 0)
                return c2

            lax.fori_loop(0, _NG, isp0, 0)
            cPA2 = pltpu.async_copy(t0_h.at[pidxt0], prowA, semC)

            cPB.wait()

            def isp1(g, c2):
                x = xv[pl.ds(g * 16, 16)]
                y = yv[pl.ds(g * 16, 16)]
                z = zv[pl.ds(g * 16, 16)]
                interp_spatial_planes(g, x, y, z, 128, prowB, 8)
                return c2

            lax.fori_loop(0, _NG, isp1, 0)
            cPB2 = pltpu.async_copy(t1_h.at[pidxt1], prowB, semD)

            cH0.wait()

            def ist0(g, c2):
                x = xv[pl.ds(g * 16, 16)]
                y = yv[pl.ds(g * 16, 16)]
                z = zv[pl.ds(g * 16, 16)]
                for l in range(0, 4):
                    interp_static_level(g, x, y, z, l, hrow0)
                return c2

            lax.fori_loop(0, _NG, ist0, 0)
            cD0 = pltpu.async_copy(hd_h.at[didx0], hrow0, semA)

            cH1.wait()

            def ist1(g, c2):
                x = xv[pl.ds(g * 16, 16)]
                y = yv[pl.ds(g * 16, 16)]
                z = zv[pl.ds(g * 16, 16)]
                for l in range(4, 8):
                    interp_static_level(g, x, y, z, l, hrow1)
                return c2

            lax.fori_loop(0, _NG, ist1, 0)
            cD1 = pltpu.async_copy(hd_h.at[didx1], hrow1, semB)

            cD0.wait()

            def idy0(g, c2):
                x = xv[pl.ds(g * 16, 16)]
                y = yv[pl.ds(g * 16, 16)]
                z = zv[pl.ds(g * 16, 16)]
                for l in range(0, 2):
                    interp_dynamic_level(g, x, y, z, l, hrow0)
                return c2

            lax.fori_loop(0, _NG, idy0, 0)
            cD2 = pltpu.async_copy(hd_h.at[didx2], hrow0, semA)

            cD1.wait()

            def idy1(g, c2):
                x = xv[pl.ds(g * 16, 16)]
                y = yv[pl.ds(g * 16, 16)]
                z = zv[pl.ds(g * 16, 16)]
                for l in range(2, 4):
                    interp_dynamic_level(g, x, y, z, l, hrow1)
                return c2

            lax.fori_loop(0, _NG, idy1, 0)
            cD3 = pltpu.async_copy(hd_h.at[didx3], hrow1, semB)

            cD2.wait()

            def idy2(g, c2):
                x = xv[pl.ds(g * 16, 16)]
                y = yv[pl.ds(g * 16, 16)]
                z = zv[pl.ds(g * 16, 16)]
                for l in range(4, 6):
                    interp_dynamic_level(g, x, y, z, l, hrow0)
                return c2

            lax.fori_loop(0, _NG, idy2, 0)

            cD3.wait()

            def idy3(g, c2):
                x = xv[pl.ds(g * 16, 16)]
                y = yv[pl.ds(g * 16, 16)]
                z = zv[pl.ds(g * 16, 16)]
                for l in range(6, 8):
                    interp_dynamic_level(g, x, y, z, l, hrow1)
                return c2

            lax.fori_loop(0, _NG, idy3, 0)

            cPA2.wait()

            def itp0(g, c2):
                x = xv[pl.ds(g * 16, 16)]
                y = yv[pl.ds(g * 16, 16)]
                z = zv[pl.ds(g * 16, 16)]
                interp_temporal_planes(g, x, y, z, 64, prowA, 16)
                return c2

            lax.fori_loop(0, _NG, itp0, 0)

            cPB2.wait()

            def itp1(g, c2):
                x = xv[pl.ds(g * 16, 16)]
                y = yv[pl.ds(g * 16, 16)]
                z = zv[pl.ds(g * 16, 16)]
                interp_temporal_planes(g, x, y, z, 128, prowB, 24)
                return c2

            lax.fori_loop(0, _NG, itp1, 0)

            pltpu.sync_copy(featb, out_h.at[pl.ds(base, ch)])
            return carry

        lax.fori_loop(0, nch, chunk_body, 0)

    return feat_kernel(xs, ys, zs, params, hsf, hdf, sp0f, sp1f, tp0f, tp1f)


def _tc_moments(feats, Wt, b2, n, interpret):
    f32 = jnp.float32
    nb = n // 1024

    def body(f_ref, wt_ref, b_ref, mean_ref, rstd_ref, accS, accV, meanv):
        p = pl.program_id(0)
        i = pl.program_id(1)

        @pl.when((p == 0) & (i == 0))
        def _init():
            accS[...] = jnp.zeros_like(accS)
            accV[...] = jnp.zeros_like(accV)

        h = lax.dot_general(
            f_ref[...], wt_ref[...], (((1,), (0,)), ((), ())),
            preferred_element_type=f32) + b_ref[...]

        @pl.when(p == 0)
        def _mean_pass():
            accS[...] += jnp.sum(h, axis=0, keepdims=True)

            @pl.when(i == nb - 1)
            def _mean_fin():
                m = accS[...] * np.float32(1.0 / n)
                meanv[...] = m
                mean_ref[...] = m

        @pl.when(p == 1)
        def _var_pass():
            d = h - meanv[...]
            accV[...] += jnp.sum(d * d, axis=0, keepdims=True)

            @pl.when(i == nb - 1)
            def _var_fin():
                var = accV[...] * np.float32(1.0 / n)
                rstd_ref[...] = 1.0 / jnp.sqrt(var + np.float32(1e-5))

    return pl.pallas_call(
        body,
        grid=(2, nb),
        in_specs=[
            pl.BlockSpec((1024, 64), lambda p, i: (i, 0)),
            pl.BlockSpec((64, 64), lambda p, i: (0, 0)),
            pl.BlockSpec((1, 64), lambda p, i: (0, 0)),
        ],
        out_specs=[
            pl.BlockSpec((1, 64), lambda p, i: (0, 0)),
            pl.BlockSpec((1, 64), lambda p, i: (0, 0)),
        ],
        out_shape=[
            jax.ShapeDtypeStruct((1, 64), f32),
            jax.ShapeDtypeStruct((1, 64), f32),
        ],
        scratch_shapes=[
            pltpu.VMEM((1, 64), f32),
            pltpu.VMEM((1, 64), f32),
            pltpu.VMEM((1, 64), f32),
        ],
        compiler_params=pltpu.CompilerParams(
            dimension_semantics=("arbitrary", "arbitrary")),
        interpret=interpret,
    )(feats, Wt, b2)


def _tc_norm(feats, Wt, b2, g2, be2, mean, rstd, n, interpret):
    f32 = jnp.float32
    nb = n // 1024

    def body(f_ref, wt_ref, b_ref, g_ref, be_ref, mean_ref, rstd_ref, o_ref):
        h = lax.dot_general(f_ref[...], wt_ref[...], (((1,), (0,)), ((), ())),
                            preferred_element_type=f32) + b_ref[...]
        o_ref[...] = jnp.maximum(
            (h - mean_ref[...]) * (rstd_ref[...] * g_ref[...]) + be_ref[...],
            0.0)

    return pl.pallas_call(
        body,
        grid=(nb,),
        in_specs=[
            pl.BlockSpec((1024, 64), lambda i: (i, 0)),
            pl.BlockSpec((64, 64), lambda i: (0, 0)),
            pl.BlockSpec((1, 64), lambda i: (0, 0)),
            pl.BlockSpec((1, 64), lambda i: (0, 0)),
            pl.BlockSpec((1, 64), lambda i: (0, 0)),
            pl.BlockSpec((1, 64), lambda i: (0, 0)),
            pl.BlockSpec((1, 64), lambda i: (0, 0)),
        ],
        out_specs=pl.BlockSpec((1024, 64), lambda i: (i, 0)),
        out_shape=jax.ShapeDtypeStruct((n, 64), f32),
        compiler_params=pltpu.CompilerParams(
            dimension_semantics=("arbitrary",)),
        interpret=interpret,
    )(feats, Wt, b2, g2, be2, mean, rstd)


def _impl(points, t, hash_s, hash_d, sp0, tp0, sp1, tp1, W, b, gamma, beta,
          interpret):
    f32 = jnp.float32
    n = points.shape[0]
    ptsT = jnp.transpose(points)
    tt = jnp.asarray(t).astype(f32) / np.float32(_FRAMES)
    params = jnp.full((16,), tt, f32)
    hsf = hash_s.reshape(_LH * _HSZ // 4, 8)
    hdf = hash_d.reshape(_LH * _HSZ // 4, 8)
    sp0f = sp0.reshape(3 * 64 * 64, 8)
    sp1f = sp1.reshape(3 * 128 * 128, 8)
    tp0f = tp0.reshape(3 * 64 * _TIME_RES, 8)
    tp1f = tp1.reshape(3 * 128 * _TIME_RES, 8)
    feats = _sc_features(ptsT[0], ptsT[1], ptsT[2], params, hsf, hdf, sp0f,
                         sp1f, tp0f, tp1f, n, interpret)
    Wt = jnp.transpose(W)
    b2 = b.reshape(1, 64)
    g2 = gamma.reshape(1, 64)
    be2 = beta.reshape(1, 64)
    mean, rstd = _tc_moments(feats, Wt, b2, n, interpret)
    return _tc_norm(feats, Wt, b2, g2, be2, mean, rstd, n, interpret)


def kernel(points, t, hash_s, hash_d, sp0, tp0, sp1, tp1, W, b, gamma, beta):
    return _impl(points, t, hash_s, hash_d, sp0, tp0, sp1, tp1, W, b, gamma,
                 beta, interpret=False)


# speedup vs baseline: 34.7247x; 2.5124x over previous
import functools
import math

import jax
import jax.numpy as jnp
import numpy as np
from jax import lax
from jax.experimental import pallas as pl
from jax.experimental.pallas import tpu as pltpu
from jax.experimental.pallas import tpu_sc as plsc

_HSZ = 524288
_MASK = _HSZ - 1
_LH = 8
_BASE, _MAXR = 16, 512
_GROWTH = math.exp(math.log(_MAXR / _BASE) / (_LH - 1))
_LEVELS = [int(math.floor(_BASE * _GROWTH ** l + 1e-6)) for l in range(_LH)]
_TIME_RES = 25
_FRAMES = 6
_P1 = np.int32(2654435761 - (1 << 32))
_P2 = np.int32(805459861)
_P3 = np.int32(3674653429 - (1 << 32))

_NW = 32
_CH = 64
_NG = _CH // 16


def _sc_features(xs, ys, zs, params, hsf, hdf, sp0f, sp1f, tp0f, tp1f, n,
                 interpret):
    ch = _CH
    pw = n // _NW
    nch = pw // ch
    f32 = jnp.float32
    i32 = jnp.int32
    mesh = plsc.VectorSubcoreMesh(core_axis_name="c", subcore_axis_name="s",
                                  num_cores=2, num_subcores=16)

    @functools.partial(
        pl.kernel,
        out_type=jax.ShapeDtypeStruct((n, 64), f32),
        mesh=mesh,
        interpret=interpret,
        compiler_params=pltpu.CompilerParams(
            needs_layout_passes=False, use_tc_tiling_on_sc=False),
        scratch_types=[
            pltpu.VMEM((ch,), f32),
            pltpu.VMEM((ch,), f32),
            pltpu.VMEM((ch,), f32),
            pltpu.VMEM((16,), f32),
            pltpu.VMEM((2048,), i32),
            pltpu.VMEM((2048,), i32),
            pltpu.VMEM((2048,), i32),
            pltpu.VMEM((2048,), i32),
            pltpu.VMEM((2048,), i32),
            pltpu.VMEM((2048,), i32),
            pltpu.VMEM((2048,), i32),
            pltpu.VMEM((2048,), i32),
            pltpu.VMEM((2048,), i32),
            pltpu.VMEM((2048,), i32),
            pltpu.VMEM((2048,), i32),
            pltpu.VMEM((2048,), i32),
            pltpu.VMEM((2048, 8), f32),
            pltpu.VMEM((2048, 8), f32),
            pltpu.VMEM((768,), i32),
            pltpu.VMEM((768,), i32),
            pltpu.VMEM((768,), i32),
            pltpu.VMEM((768,), i32),
            pltpu.VMEM((768, 8), f32),
            pltpu.VMEM((768, 8), f32),
            pltpu.VMEM((ch, 64), f32),
            pltpu.SemaphoreType.DMA,
            pltpu.SemaphoreType.DMA,
            pltpu.SemaphoreType.DMA,
            pltpu.SemaphoreType.DMA,
        ],
    )
    def feat_kernel(xs_h, ys_h, zs_h, par_h, hs_h, hd_h, s0_h, s1_h, t0_h,
                    t1_h, out_h,
                    xv, yv, zv, pv, sidx0, sidx1, sidx2, sidx3,
                    didx0, didx1, didx2, didx3, didx4, didx5, didx6, didx7,
                    hrow0, hrow1, pidx0, pidx1, pidxt0, pidxt1, prowA, prowB,
                    featb, semA, semB, semC, semD):
        wid = lax.axis_index("s") * 2 + lax.axis_index("c")
        lanes = lax.iota(i32, 16)
        zeros_i = jnp.zeros((16,), i32)
        ones_i = jnp.full((16,), 1, i32)

        pltpu.sync_copy(par_h, pv)
        tv = pv[...]
        pt = tv * np.float32(_TIME_RES)
        ti = pt.astype(i32)
        ftv = pt - ti.astype(f32)
        wt0 = 1.0 - ftv
        wt1 = ftv
        ht0 = ti * _P3
        ht1 = ht0 + _P3
        bt = tv * np.float32(_TIME_RES - 1)
        bti = jnp.minimum(jnp.maximum(bt.astype(i32), 0),
                          np.int32(_TIME_RES - 2))
        fbt = bt - bti.astype(f32)
        wbt0 = 1.0 - fbt
        wbt1 = fbt

        sidxs = (sidx0, sidx1, sidx2, sidx3)
        didxs = (didx0, didx1, didx2, didx3, didx4, didx5, didx6, didx7)

        def phys_row(h19, l):
            return (((h19 >> 7) << 5) + ((h19 >> 3) & 15)
                    + np.int32(l * 131072))

        def cfrac(c, rm1f, rmax):
            p = c * rm1f
            ci = jnp.minimum(p.astype(i32), np.int32(rmax))
            ci = jnp.maximum(ci, np.int32(0))
            return ci, p - ci.astype(f32)

        def gen_spatial_planes(g, x, y, z, R, pref):
            rm1 = np.float32(R - 1)
            xi, _ = cfrac(x, rm1, R - 2)
            yi, _ = cfrac(y, rm1, R - 2)
            zi, _ = cfrac(z, rm1, R - 2)
            coords = ((xi, yi), (xi, zi), (yi, zi))
            for k in range(3):
                ai, bi = coords[k]
                off = np.int32(k * R * R)
                for da in (0, 1):
                    for db in (0, 1):
                        cor = da * 2 + db
                        row = (ai + da) * np.int32(R) + (bi + db) + off
                        pref[pl.ds(k * 256 + cor * 64 + g * 16, 16)] = row

        def gen_temporal_planes(g, x, y, z, R1, pref):
            rm1 = np.float32(R1 - 1)
            coords = (x, y, z)
            for k in range(3):
                ai, _ = cfrac(coords[k], rm1, R1 - 2)
                off = np.int32(k * R1 * _TIME_RES)
                for da in (0, 1):
                    for db in (0, 1):
                        cor = da * 2 + db
                        row = ((ai + da) * np.int32(_TIME_RES)
                               + (bti + db) + off)
                        pref[pl.ds(k * 256 + cor * 64 + g * 16, 16)] = row

        def interp_spatial_planes(g, x, y, z, R, prow, col0):
            rm1 = np.float32(R - 1)
            _, fx = cfrac(x, rm1, R - 2)
            _, fy = cfrac(y, rm1, R - 2)
            _, fz = cfrac(z, rm1, R - 2)
            fr = (fx, fy, fz)
            pairs = ((0, 1), (0, 2), (1, 2))
            ws = []
            for k in range(3):
                fa, fb = fr[pairs[k][0]], fr[pairs[k][1]]
                ws.append(((1.0 - fa) * (1.0 - fb), (1.0 - fa) * fb,
                           fa * (1.0 - fb), fa * fb))
            rowv = g * 16 + lanes
            for chn in range(8):
                cv = jnp.full((16,), chn, i32)
                prod = None
                for k in range(3):
                    acc = None
                    for cor in range(4):
                        ridx = k * 256 + cor * 64 + g * 16 + lanes
                        fv = plsc.load_gather(prow, [ridx, cv])
                        term = ws[k][cor] * fv
                        acc = term if acc is None else acc + term
                    prod = acc if prod is None else prod * acc
                plsc.store_scatter(
                    featb, [rowv, jnp.full((16,), col0 + chn, i32)], prod)

        def interp_temporal_planes(g, x, y, z, R1, prow, col0):
            rm1 = np.float32(R1 - 1)
            coords = (x, y, z)
            ws = []
            for k in range(3):
                _, fa = cfrac(coords[k], rm1, R1 - 2)
                ws.append(((1.0 - fa) * wbt0, (1.0 - fa) * wbt1,
                           fa * wbt0, fa * wbt1))
            rowv = g * 16 + lanes
            for chn in range(8):
                cv = jnp.full((16,), chn, i32)
                prod = None
                for k in range(3):
                    acc = None
                    for cor in range(4):
                        ridx = k * 256 + cor * 64 + g * 16 + lanes
                        fv = plsc.load_gather(prow, [ridx, cv])
                        term = ws[k][cor] * fv
                        acc = term if acc is None else acc + term
                    prod = acc if prod is None else prod * acc
                plsc.store_scatter(
                    featb, [rowv, jnp.full((16,), col0 + chn, i32)], prod)

        def corner_hashes(x, y, z, l):
            r = np.float32(_LEVELS[l])
            px = x * r
            py = y * r
            pz = z * r
            xi = px.astype(i32)
            yi = py.astype(i32)
            zi = pz.astype(i32)
            fx = px - xi.astype(f32)
            fy = py - yi.astype(f32)
            fz = pz - zi.astype(f32)
            hy0 = yi * _P1
            hz0 = zi * _P2
            hxs = (xi, xi + 1)
            hys = (hy0, hy0 + _P1)
            hzs = (hz0, hz0 + _P2)
            hsps = []
            for dx in (0, 1):
                for dy in (0, 1):
                    hxy = hxs[dx] ^ hys[dy]
                    for dz in (0, 1):
                        hsps.append(hxy ^ hzs[dz])
            return hsps, (1.0 - fx, fx), (1.0 - fy, fy), (1.0 - fz, fz)

        def interp_static_level(g, x, y, z, l, hrow):
            hsps, wxs, wys, wzs = corner_hashes(x, y, z, l)
            rowv = g * 16 + lanes
            acc0 = jnp.zeros((16,), f32)
            acc1 = jnp.zeros((16,), f32)
            for dx in (0, 1):
                for dy in (0, 1):
                    wxy = wxs[dx] * wys[dy]
                    for dz in (0, 1):
                        ci = dx * 4 + dy * 2 + dz
                        ridx = (l % 2) * 1024 + g * 128 + ci * 16 + lanes
                        col0 = hsps[ci] & 7
                        f0 = plsc.load_gather(hrow, [ridx, col0])
                        f1 = plsc.load_gather(hrow, [ridx + 512, col0])
                        w = wxy * wzs[dz]
                        acc0 = acc0 + w * f0
                        acc1 = acc1 + w * f1
            plsc.store_scatter(featb, [rowv, jnp.full((16,), 32 + 2 * l, i32)],
                               acc0)
            plsc.store_scatter(featb, [rowv, jnp.full((16,), 33 + 2 * l, i32)],
                               acc1)

        def interp_dynamic_level(g, x, y, z, l, hrow):
            hsps, wxs, wys, wzs = corner_hashes(x, y, z, l)
            rowv = g * 16 + lanes
            acc0 = jnp.zeros((16,), f32)
            acc1 = jnp.zeros((16,), f32)
            for dx in (0, 1):
                for dy in (0, 1):
                    wxy = wxs[dx] * wys[dy]
                    for dz in (0, 1):
                        ci = dx * 4 + dy * 2 + dz
                        r0 = g * 128 + ci * 16 + lanes
                        c0 = (hsps[ci] ^ ht0) & 7
                        c1 = (hsps[ci] ^ ht1) & 7
                        fa0 = plsc.load_gather(hrow, [r0, c0])
                        fa1 = plsc.load_gather(hrow, [r0 + 512, c0])
                        fb0 = plsc.load_gather(hrow, [r0 + 1024, c1])
                        fb1 = plsc.load_gather(hrow, [r0 + 1536, c1])
                        w = wxy * wzs[dz]
                        acc0 = acc0 + w * (wt0 * fa0 + wt1 * fb0)
                        acc1 = acc1 + w * (wt0 * fa1 + wt1 * fb1)
            plsc.store_scatter(featb, [rowv, jnp.full((16,), 48 + 2 * l, i32)],
                               acc0)
            plsc.store_scatter(featb, [rowv, jnp.full((16,), 49 + 2 * l, i32)],
                               acc1)

        def chunk_body(cidx, carry):
            base = wid * pw + cidx * ch
            pltpu.sync_copy(xs_h.at[pl.ds(base, ch)], xv)
            pltpu.sync_copy(ys_h.at[pl.ds(base, ch)], yv)
            pltpu.sync_copy(zs_h.at[pl.ds(base, ch)], zv)

            def gen(g, c2):
                x = xv[pl.ds(g * 16, 16)]
                y = yv[pl.ds(g * 16, 16)]
                z = zv[pl.ds(g * 16, 16)]
                for l in range(_LH):
                    r = np.float32(_LEVELS[l])
                    xi = (x * r).astype(i32)
                    yi = (y * r).astype(i32)
                    zi = (z * r).astype(i32)
                    hxs = (xi, xi + 1)
                    hy0 = yi * _P1
                    hys = (hy0, hy0 + _P1)
                    hz0 = zi * _P2
                    hzs = (hz0, hz0 + _P2)
                    sref = sidxs[l // 2]
                    sbase = (l % 2) * 1024 + g * 128
                    dref = didxs[l]
                    dbase = g * 128
                    for dx in (0, 1):
                        for dy in (0, 1):
                            hxy = hxs[dx] ^ hys[dy]
                            for dz in (0, 1):
                                ci = dx * 4 + dy * 2 + dz
                                hsp = hxy ^ hzs[dz]
                                rs = phys_row(hsp & _MASK, l)
                                sref[pl.ds(sbase + ci * 16, 16)] = rs
                                sref[pl.ds(sbase + 512 + ci * 16, 16)] = rs + 16
                                rd0 = phys_row((hsp ^ ht0) & _MASK, l)
                                dref[pl.ds(dbase + ci * 16, 16)] = rd0
                                dref[pl.ds(dbase + 512 + ci * 16, 16)] = rd0 + 16
                                rd1 = phys_row((hsp ^ ht1) & _MASK, l)
                                dref[pl.ds(dbase + 1024 + ci * 16, 16)] = rd1
                                dref[pl.ds(dbase + 1536 + ci * 16, 16)] = rd1 + 16
                gen_spatial_planes(g, x, y, z, 64, pidx0)
                gen_spatial_planes(g, x, y, z, 128, pidx1)
                gen_temporal_planes(g, x, y, z, 64, pidxt0)
                gen_temporal_planes(g, x, y, z, 128, pidxt1)
                return c2

            lax.fori_loop(0, _NG, gen, 0)

            cPA = pltpu.async_copy(s0_h.at[pidx0], prowA, semC)
            cPB = pltpu.async_copy(s1_h.at[pidx1], prowB, semD)
            hbufs = (hrow0, hrow1)
            hsems = (semA, semB)

            def fire_sub(j):
                if j < 4:
                    return pltpu.async_copy(hs_h.at[sidxs[j]],
                                            hbufs[j % 2], hsems[j % 2])
                return pltpu.async_copy(hd_h.at[didxs[j - 4]],
                                        hbufs[j % 2], hsems[j % 2])

            hhandles = [fire_sub(0), fire_sub(1)]

            cPA.wait()

            def isp0(g, c2):
                x = xv[pl.ds(g * 16, 16)]
                y = yv[pl.ds(g * 16, 16)]
                z = zv[pl.ds(g * 16, 16)]
                interp_spatial_planes(g, x, y, z, 64, prowA, 0)
                return c2

            lax.fori_loop(0, _NG, isp0, 0)
            cPA2 = pltpu.async_copy(t0_h.at[pidxt0], prowA, semC)

            cPB.wait()

            def isp1(g, c2):
                x = xv[pl.ds(g * 16, 16)]
                y = yv[pl.ds(g * 16, 16)]
                z = zv[pl.ds(g * 16, 16)]
                interp_spatial_planes(g, x, y, z, 128, prowB, 8)
                return c2

            lax.fori_loop(0, _NG, isp1, 0)
            cPB2 = pltpu.async_copy(t1_h.at[pidxt1], prowB, semD)

            def make_interp_sub(j, buf):
                def isub(g, c2):
                    x = xv[pl.ds(g * 16, 16)]
                    y = yv[pl.ds(g * 16, 16)]
                    z = zv[pl.ds(g * 16, 16)]
                    if j < 4:
                        interp_static_level(g, x, y, z, 2 * j, buf)
                        interp_static_level(g, x, y, z, 2 * j + 1, buf)
                    else:
                        interp_dynamic_level(g, x, y, z, j - 4, buf)
                    return c2
                return isub

            for j in range(12):
                hhandles[j].wait()
                lax.fori_loop(0, _NG, make_interp_sub(j, hbufs[j % 2]), 0)
                if j + 2 < 12:
                    hhandles.append(fire_sub(j + 2))

            cPA2.wait()

            def itp0(g, c2):
                x = xv[pl.ds(g * 16, 16)]
                y = yv[pl.ds(g * 16, 16)]
                z = zv[pl.ds(g * 16, 16)]
                interp_temporal_planes(g, x, y, z, 64, prowA, 16)
                return c2

            lax.fori_loop(0, _NG, itp0, 0)

            cPB2.wait()

            def itp1(g, c2):
                x = xv[pl.ds(g * 16, 16)]
                y = yv[pl.ds(g * 16, 16)]
                z = zv[pl.ds(g * 16, 16)]
                interp_temporal_planes(g, x, y, z, 128, prowB, 24)
                return c2

            lax.fori_loop(0, _NG, itp1, 0)

            pltpu.sync_copy(featb, out_h.at[pl.ds(base, ch)])
            return carry

        lax.fori_loop(0, nch, chunk_body, 0)

    return feat_kernel(xs, ys, zs, params, hsf, hdf, sp0f, sp1f, tp0f, tp1f)


def _tc_moments(feats, Wt, b2, n, interpret):
    f32 = jnp.float32
    nb = n // 1024

    def body(f_ref, wt_ref, b_ref, mean_ref, rstd_ref, accS, accV, meanv):
        p = pl.program_id(0)
        i = pl.program_id(1)

        @pl.when((p == 0) & (i == 0))
        def _init():
            accS[...] = jnp.zeros_like(accS)
            accV[...] = jnp.zeros_like(accV)

        h = lax.dot_general(
            f_ref[...], wt_ref[...], (((1,), (0,)), ((), ())),
            preferred_element_type=f32) + b_ref[...]

        @pl.when(p == 0)
        def _mean_pass():
            accS[...] += jnp.sum(h, axis=0, keepdims=True)

            @pl.when(i == nb - 1)
            def _mean_fin():
                m = accS[...] * np.float32(1.0 / n)
                meanv[...] = m
                mean_ref[...] = m

        @pl.when(p == 1)
        def _var_pass():
            d = h - meanv[...]
            accV[...] += jnp.sum(d * d, axis=0, keepdims=True)

            @pl.when(i == nb - 1)
            def _var_fin():
                var = accV[...] * np.float32(1.0 / n)
                rstd_ref[...] = 1.0 / jnp.sqrt(var + np.float32(1e-5))

    return pl.pallas_call(
        body,
        grid=(2, nb),
        in_specs=[
            pl.BlockSpec((1024, 64), lambda p, i: (i, 0)),
            pl.BlockSpec((64, 64), lambda p, i: (0, 0)),
            pl.BlockSpec((1, 64), lambda p, i: (0, 0)),
        ],
        out_specs=[
            pl.BlockSpec((1, 64), lambda p, i: (0, 0)),
            pl.BlockSpec((1, 64), lambda p, i: (0, 0)),
        ],
        out_shape=[
            jax.ShapeDtypeStruct((1, 64), f32),
            jax.ShapeDtypeStruct((1, 64), f32),
        ],
        scratch_shapes=[
            pltpu.VMEM((1, 64), f32),
            pltpu.VMEM((1, 64), f32),
            pltpu.VMEM((1, 64), f32),
        ],
        compiler_params=pltpu.CompilerParams(
            dimension_semantics=("arbitrary", "arbitrary")),
        interpret=interpret,
    )(feats, Wt, b2)


def _tc_norm(feats, Wt, b2, g2, be2, mean, rstd, n, interpret):
    f32 = jnp.float32
    nb = n // 1024

    def body(f_ref, wt_ref, b_ref, g_ref, be_ref, mean_ref, rstd_ref, o_ref):
        h = lax.dot_general(f_ref[...], wt_ref[...], (((1,), (0,)), ((), ())),
                            preferred_element_type=f32) + b_ref[...]
        o_ref[...] = jnp.maximum(
            (h - mean_ref[...]) * (rstd_ref[...] * g_ref[...]) + be_ref[...],
            0.0)

    return pl.pallas_call(
        body,
        grid=(nb,),
        in_specs=[
            pl.BlockSpec((1024, 64), lambda i: (i, 0)),
            pl.BlockSpec((64, 64), lambda i: (0, 0)),
            pl.BlockSpec((1, 64), lambda i: (0, 0)),
            pl.BlockSpec((1, 64), lambda i: (0, 0)),
            pl.BlockSpec((1, 64), lambda i: (0, 0)),
            pl.BlockSpec((1, 64), lambda i: (0, 0)),
            pl.BlockSpec((1, 64), lambda i: (0, 0)),
        ],
        out_specs=pl.BlockSpec((1024, 64), lambda i: (i, 0)),
        out_shape=jax.ShapeDtypeStruct((n, 64), f32),
        compiler_params=pltpu.CompilerParams(
            dimension_semantics=("arbitrary",)),
        interpret=interpret,
    )(feats, Wt, b2, g2, be2, mean, rstd)


def _impl(points, t, hash_s, hash_d, sp0, tp0, sp1, tp1, W, b, gamma, beta,
          interpret):
    f32 = jnp.float32
    n = points.shape[0]
    ptsT = jnp.transpose(points)
    tt = jnp.asarray(t).astype(f32) / np.float32(_FRAMES)
    params = jnp.full((16,), tt, f32)
    hsf = hash_s.reshape(_LH, 4096, 128, 2).transpose(0, 1, 3, 2).reshape(
        _LH * _HSZ // 4, 8)
    hdf = hash_d.reshape(_LH, 4096, 128, 2).transpose(0, 1, 3, 2).reshape(
        _LH * _HSZ // 4, 8)
    sp0f = sp0.reshape(3 * 64 * 64, 8)
    sp1f = sp1.reshape(3 * 128 * 128, 8)
    tp0f = tp0.reshape(3 * 64 * _TIME_RES, 8)
    tp1f = tp1.reshape(3 * 128 * _TIME_RES, 8)
    feats = _sc_features(ptsT[0], ptsT[1], ptsT[2], params, hsf, hdf, sp0f,
                         sp1f, tp0f, tp1f, n, interpret)
    Wt = jnp.transpose(W)
    b2 = b.reshape(1, 64)
    g2 = gamma.reshape(1, 64)
    be2 = beta.reshape(1, 64)
    mean, rstd = _tc_moments(feats, Wt, b2, n, interpret)
    return _tc_norm(feats, Wt, b2, g2, be2, mean, rstd, n, interpret)


def kernel(points, t, hash_s, hash_d, sp0, tp0, sp1, tp1, W, b, gamma, beta):
    return _impl(points, t, hash_s, hash_d, sp0, tp0, sp1, tp1, W, b, gamma,
                 beta, interpret=False)
